# Initial kernel scaffold; baseline (speedup 1.0000x reference)
import jax
import jax.numpy as jnp
from jax.experimental import pallas as pl


def kernel(x_drug, x_target, edge_index_dd, edge_index_tt, W_drug, b_drug, W_target, b_target):
    raise NotImplementedError("write your pallas kernel here")


# speedup vs baseline: 3.9485x; 3.9485x over previous
import jax
import jax.numpy as jnp
from jax import lax
from jax.experimental import pallas as pl
from jax.experimental.pallas import tpu as pltpu
from jax.experimental.pallas import tpu_sc as plsc

N = 10000
D = 128
E = 320000
NS = 16
L = 16
ROWS_PER_TILE = 624
TAIL_ROWS = N - NS * ROWS_PER_TILE
BLOCK_ROWS = 208
NBLK = ROWS_PER_TILE // BLOCK_ROWS
EDGES_PER_TILE = E // NS
CHUNK = 80
NCHUNK = EDGES_PER_TILE // CHUNK


def _sc_body(x_d, x_t, src_d, dst_d, src_t, dst_t, z128,
             mean_d_out, mean_t_out,
             src_v, dst_v, rows_v, node_v, rcp_v, acc_s, sem):
    c = lax.axis_index("c")
    s = lax.axis_index("s")
    row0 = s * ROWS_PER_TILE
    is_tail = s == NS - 1

    ones16 = jnp.full((L,), 1.0, jnp.float32)

    @pl.loop(0, CHUNK)
    def _(i):
        for k in range(D // L):
            rows_v[i, pl.ds(k * L, L)] = ones16

    def zero_acc():
        pltpu.sync_copy(z128, acc_s.at[pl.ds(row0, ROWS_PER_TILE)])

        @pl.when(is_tail)
        def _():
            pltpu.sync_copy(z128.at[pl.ds(0, TAIL_ROWS)],
                            acc_s.at[pl.ds(NS * ROWS_PER_TILE, TAIL_ROWS)])

    zero_acc()
    plsc.subcore_barrier()

    def scan_edges(dst_hbm, body):
        base_t = s * EDGES_PER_TILE

        @pl.loop(0, NCHUNK)
        def _(i):
            base = base_t + i * CHUNK
            pltpu.sync_copy(dst_hbm.at[pl.ds(base, CHUNK)], dst_v)
            body(base)

    def phase_a(dst_hbm):
        scan_edges(dst_hbm, lambda base: pltpu.sync_copy(
            rows_v, acc_s.at[dst_v], add=True))

    @pl.when(c == 0)
    def _():
        phase_a(dst_d)

    @pl.when(c == 1)
    def _():
        phase_a(dst_t)

    plsc.subcore_barrier()

    def keep_rcp(nrows, src_row, dst_el):
        pltpu.sync_copy(acc_s.at[pl.ds(src_row, nrows)],
                        node_v.at[pl.ds(0, nrows)])

        @pl.loop(0, nrows)
        def _(i):
            deg = node_v[i, pl.ds(0, L)]
            rcp_v[pl.ds(dst_el + i * L, L)] = 1.0 / jnp.maximum(deg, 1.0)

    for b in range(NBLK):
        keep_rcp(BLOCK_ROWS, row0 + b * BLOCK_ROWS, b * BLOCK_ROWS * L)

    @pl.when(is_tail)
    def _():
        keep_rcp(TAIL_ROWS, NS * ROWS_PER_TILE, ROWS_PER_TILE * L)

    plsc.subcore_barrier()
    zero_acc()
    plsc.subcore_barrier()

    def phase_b(x_hbm, src_hbm, dst_hbm):
        base_t = s * EDGES_PER_TILE

        @pl.loop(0, NCHUNK)
        def _(i):
            base = base_t + i * CHUNK
            pltpu.sync_copy(src_hbm.at[pl.ds(base, CHUNK)], src_v)
            pltpu.sync_copy(dst_hbm.at[pl.ds(base, CHUNK)], dst_v)
            pltpu.async_copy(x_hbm.at[src_v], rows_v, sem).wait()
            pltpu.sync_copy(rows_v, acc_s.at[dst_v], add=True)

    @pl.when(c == 0)
    def _():
        phase_b(x_d, src_d, dst_d)

    @pl.when(c == 1)
    def _():
        phase_b(x_t, src_t, dst_t)

    plsc.subcore_barrier()

    def flush_block(mean_out, nrows, src_row, rcp_el):
        pltpu.sync_copy(acc_s.at[pl.ds(src_row, nrows)],
                        node_v.at[pl.ds(0, nrows)])

        @pl.loop(0, nrows)
        def _(i):
            rcp = rcp_v[pl.ds(rcp_el + i * L, L)]
            for k in range(D // L):
                node_v[i, pl.ds(k * L, L)] = node_v[i, pl.ds(k * L, L)] * rcp

        pltpu.sync_copy(node_v.at[pl.ds(0, nrows)],
                        mean_out.at[pl.ds(src_row, nrows)])

    def flush(mean_out):
        for b in range(NBLK):
            flush_block(mean_out, BLOCK_ROWS, row0 + b * BLOCK_ROWS,
                        b * BLOCK_ROWS * L)

        @pl.when(is_tail)
        def _():
            flush_block(mean_out, TAIL_ROWS, NS * ROWS_PER_TILE,
                        ROWS_PER_TILE * L)

    @pl.when(c == 0)
    def _():
        flush(mean_d_out)

    @pl.when(c == 1)
    def _():
        flush(mean_t_out)


_sc_aggregate = pl.kernel(
    _sc_body,
    out_type=(
        jax.ShapeDtypeStruct((N, D), jnp.float32),
        jax.ShapeDtypeStruct((N, D), jnp.float32),
    ),
    mesh=plsc.VectorSubcoreMesh(core_axis_name="c", subcore_axis_name="s"),
    scratch_types=[
        pltpu.VMEM((CHUNK,), jnp.int32),
        pltpu.VMEM((CHUNK,), jnp.int32),
        pltpu.VMEM((CHUNK, D), jnp.float32),
        pltpu.VMEM((BLOCK_ROWS, D), jnp.float32),
        pltpu.VMEM(((ROWS_PER_TILE + TAIL_ROWS) * L,), jnp.float32),
        pltpu.VMEM_SHARED((N, D), jnp.float32),
        pltpu.SemaphoreType.DMA,
    ],
)


def _finish_body(x_ref, mean_ref, m_ref, w_ref, b_ref, o_ref):
    h = jnp.dot(mean_ref[...], w_ref[...],
                preferred_element_type=jnp.float32) + b_ref[...]
    h = jnp.maximum(h, 0.0)
    o_ref[...] = jnp.where(m_ref[...] != 0.0, x_ref[...], h)


def _finish(x, mean, m1, W, b2):
    R = 1000
    return pl.pallas_call(
        _finish_body,
        grid=(N // R,),
        in_specs=[
            pl.BlockSpec((R, D), lambda i: (i, 0)),
            pl.BlockSpec((R, D), lambda i: (i, 0)),
            pl.BlockSpec((R, 1), lambda i: (i, 0)),
            pl.BlockSpec((D, D), lambda i: (0, 0)),
            pl.BlockSpec((1, D), lambda i: (0, 0)),
        ],
        out_specs=pl.BlockSpec((R, D), lambda i: (i, 0)),
        out_shape=jax.ShapeDtypeStruct((N, D), jnp.float32),
    )(x, mean, m1, W, b2)


def kernel(x_drug, x_target, edge_index_dd, edge_index_tt,
           W_drug, b_drug, W_target, b_target):
    src_d = edge_index_dd[0]
    dst_d = edge_index_dd[1]
    src_t = edge_index_tt[0]
    dst_t = edge_index_tt[1]

    z128 = jnp.zeros((ROWS_PER_TILE, D), jnp.float32)

    mean_d, mean_t = _sc_aggregate(
        x_drug, x_target, src_d, dst_d, src_t, dst_t, z128)

    u_d = jax.random.uniform(jax.random.key(42), (N,), dtype=jnp.float32)
    u_t = jax.random.uniform(jax.random.key(43), (N,), dtype=jnp.float32)
    m_d = (u_d < 0.5).astype(jnp.float32).reshape(N, 1)
    m_t = (u_t < 0.5).astype(jnp.float32).reshape(N, 1)

    z_drug = _finish(x_drug, mean_d, m_d, W_drug, b_drug.reshape(1, D))
    z_target = _finish(x_target, mean_t, m_t, W_target, b_target.reshape(1, D))
    return (z_drug, z_target)


# speedup vs baseline: 7.2522x; 1.8367x over previous
import jax
import jax.numpy as jnp
from jax import lax
from jax.experimental import pallas as pl
from jax.experimental.pallas import tpu as pltpu
from jax.experimental.pallas import tpu_sc as plsc

N = 10000
D = 128
E = 320000
NS = 16
L = 16
ROWS_PER_TILE = 624
TAIL_ROWS = N - NS * ROWS_PER_TILE
CHUNK = 128
NROWS = E // CHUNK
BPC = 8
NFULL = 19

BLOCKS = ((0, 128), (128, 128), (256, 128), (384, 128), (512, 112))


def _sc_body(x_d, x_t, src_d, dst_d, src_t, dst_t, z128,
             mean_d_out, mean_t_out,
             si, di, rb0, rb1, rcp_v, acc_s, gs0, gs1):
    c = lax.axis_index("c")
    s = lax.axis_index("s")
    row0 = s * ROWS_PER_TILE
    is_tail = s == NS - 1
    rb = (rb0, rb1)
    gs = (gs0, gs1)

    ones16 = jnp.full((L,), 1.0, jnp.float32)

    @pl.loop(0, CHUNK)
    def _(i):
        for k in range(D // L):
            rb0[i, pl.ds(k * L, L)] = ones16

    def zero_acc():
        pltpu.sync_copy(z128, acc_s.at[pl.ds(row0, ROWS_PER_TILE)])

        @pl.when(is_tail)
        def _():
            pltpu.sync_copy(z128.at[pl.ds(0, TAIL_ROWS)],
                            acc_s.at[pl.ds(NS * ROWS_PER_TILE, TAIL_ROWS)])

    zero_acc()
    plsc.subcore_barrier()

    def batch_a(dst2, i, nchunks):
        row = 8 * s + (BPC * NS) * i
        pltpu.sync_copy(dst2.at[pl.ds(row, nchunks)], di.at[pl.ds(0, nchunks)])
        for j in range(nchunks):
            pltpu.sync_copy(rb0, acc_s.at[di.at[j]], add=True)

    def phase_a(dst2):
        @pl.loop(0, NFULL)
        def _(i):
            batch_a(dst2, i, BPC)

        @pl.when(s < 8)
        def _():
            batch_a(dst2, NFULL, BPC)

        @pl.when(s == 8)
        def _():
            batch_a(dst2, NFULL, 4)

    @pl.when(c == 0)
    def _():
        phase_a(dst_d)

    @pl.when(c == 1)
    def _():
        phase_a(dst_t)

    plsc.subcore_barrier()

    def keep_rcp(nrows, src_row, dst_el):
        pltpu.sync_copy(acc_s.at[pl.ds(src_row, nrows)],
                        rb1.at[pl.ds(0, nrows)])

        @pl.loop(0, nrows)
        def _(i):
            deg = rb1[i, pl.ds(0, L)]
            rcp_v[pl.ds(dst_el + i * L, L)] = 1.0 / jnp.maximum(deg, 1.0)

    for local, nrows in BLOCKS:
        keep_rcp(nrows, row0 + local, local * L)

    @pl.when(is_tail)
    def _():
        keep_rcp(TAIL_ROWS, NS * ROWS_PER_TILE, ROWS_PER_TILE * L)

    plsc.subcore_barrier()
    zero_acc()
    plsc.subcore_barrier()

    def batch_b(x_hbm, src2, dst2, i, nchunks):
        row = 8 * s + (BPC * NS) * i
        pltpu.sync_copy(src2.at[pl.ds(row, nchunks)], si.at[pl.ds(0, nchunks)])
        pltpu.sync_copy(dst2.at[pl.ds(row, nchunks)], di.at[pl.ds(0, nchunks)])
        copies = [None] * nchunks
        copies[0] = pltpu.async_copy(x_hbm.at[si.at[0]], rb[0], gs[0])
        for j in range(nchunks):
            copies[j].wait()
            if j + 1 < nchunks:
                copies[j + 1] = pltpu.async_copy(
                    x_hbm.at[si.at[j + 1]], rb[(j + 1) % 2], gs[(j + 1) % 2])
            pltpu.sync_copy(rb[j % 2], acc_s.at[di.at[j]], add=True)

    def phase_b(x_hbm, src2, dst2):
        @pl.loop(0, NFULL)
        def _(i):
            batch_b(x_hbm, src2, dst2, i, BPC)

        @pl.when(s < 8)
        def _():
            batch_b(x_hbm, src2, dst2, NFULL, BPC)

        @pl.when(s == 8)
        def _():
            batch_b(x_hbm, src2, dst2, NFULL, 4)

    @pl.when(c == 0)
    def _():
        phase_b(x_d, src_d, dst_d)

    @pl.when(c == 1)
    def _():
        phase_b(x_t, src_t, dst_t)

    plsc.subcore_barrier()

    def flush_block(mean_out, nrows, src_row, rcp_el):
        pltpu.sync_copy(acc_s.at[pl.ds(src_row, nrows)],
                        rb0.at[pl.ds(0, nrows)])

        @pl.loop(0, nrows)
        def _(i):
            rcp = rcp_v[pl.ds(rcp_el + i * L, L)]
            for k in range(D // L):
                rb0[i, pl.ds(k * L, L)] = rb0[i, pl.ds(k * L, L)] * rcp

        pltpu.sync_copy(rb0.at[pl.ds(0, nrows)],
                        mean_out.at[pl.ds(src_row, nrows)])

    def flush(mean_out):
        for local, nrows in BLOCKS:
            flush_block(mean_out, nrows, row0 + local, local * L)

        @pl.when(is_tail)
        def _():
            flush_block(mean_out, TAIL_ROWS, NS * ROWS_PER_TILE,
                        ROWS_PER_TILE * L)

    @pl.when(c == 0)
    def _():
        flush(mean_d_out)

    @pl.when(c == 1)
    def _():
        flush(mean_t_out)


_sc_aggregate = pl.kernel(
    _sc_body,
    out_type=(
        jax.ShapeDtypeStruct((N, D), jnp.float32),
        jax.ShapeDtypeStruct((N, D), jnp.float32),
    ),
    mesh=plsc.VectorSubcoreMesh(core_axis_name="c", subcore_axis_name="s"),
    scratch_types=[
        pltpu.VMEM((BPC, CHUNK), jnp.int32),
        pltpu.VMEM((BPC, CHUNK), jnp.int32),
        pltpu.VMEM((CHUNK, D), jnp.float32),
        pltpu.VMEM((CHUNK, D), jnp.float32),
        pltpu.VMEM(((ROWS_PER_TILE + TAIL_ROWS) * L,), jnp.float32),
        pltpu.VMEM_SHARED((N, D), jnp.float32),
        pltpu.SemaphoreType.DMA,
        pltpu.SemaphoreType.DMA,
    ],
)


def _finish_body(x_ref, mean_ref, m_ref, w_ref, b_ref, o_ref):
    h = jnp.dot(mean_ref[...], w_ref[...],
                preferred_element_type=jnp.float32) + b_ref[...]
    h = jnp.maximum(h, 0.0)
    o_ref[...] = jnp.where(m_ref[...] != 0.0, x_ref[...], h)


def _finish(x, mean, m1, W, b2):
    R = 1000
    return pl.pallas_call(
        _finish_body,
        grid=(N // R,),
        in_specs=[
            pl.BlockSpec((R, D), lambda i: (i, 0)),
            pl.BlockSpec((R, D), lambda i: (i, 0)),
            pl.BlockSpec((R, 1), lambda i: (i, 0)),
            pl.BlockSpec((D, D), lambda i: (0, 0)),
            pl.BlockSpec((1, D), lambda i: (0, 0)),
        ],
        out_specs=pl.BlockSpec((R, D), lambda i: (i, 0)),
        out_shape=jax.ShapeDtypeStruct((N, D), jnp.float32),
    )(x, mean, m1, W, b2)


def kernel(x_drug, x_target, edge_index_dd, edge_index_tt,
           W_drug, b_drug, W_target, b_target):
    src_d = edge_index_dd[0].reshape(NROWS, CHUNK)
    dst_d = edge_index_dd[1].reshape(NROWS, CHUNK)
    src_t = edge_index_tt[0].reshape(NROWS, CHUNK)
    dst_t = edge_index_tt[1].reshape(NROWS, CHUNK)

    z128 = jnp.zeros((ROWS_PER_TILE, D), jnp.float32)

    mean_d, mean_t = _sc_aggregate(
        x_drug, x_target, src_d, dst_d, src_t, dst_t, z128)

    u_d = jax.random.uniform(jax.random.key(42), (N,), dtype=jnp.float32)
    u_t = jax.random.uniform(jax.random.key(43), (N,), dtype=jnp.float32)
    m_d = (u_d < 0.5).astype(jnp.float32).reshape(N, 1)
    m_t = (u_t < 0.5).astype(jnp.float32).reshape(N, 1)

    z_drug = _finish(x_drug, mean_d, m_d, W_drug, b_drug.reshape(1, D))
    z_target = _finish(x_target, mean_t, m_t, W_target, b_target.reshape(1, D))
    return (z_drug, z_target)


# speedup vs baseline: 7.2810x; 1.0040x over previous
import jax
import jax.numpy as jnp
from jax import lax
from jax.experimental import pallas as pl
from jax.experimental.pallas import tpu as pltpu
from jax.experimental.pallas import tpu_sc as plsc

N = 10000
D = 128
E = 320000
NS = 16
L = 16
ROWS_PER_TILE = 624
TAIL_ROWS = N - NS * ROWS_PER_TILE
CHUNK = 128
NROWS = E // CHUNK
BPC = 8
NFULL = 19

BLOCKS = ((0, 128), (128, 128), (256, 128), (384, 128), (512, 112))


def _sc_body(x_d, x_t, src_d, dst_d, src_t, dst_t, z128,
             mean_d_out, mean_t_out,
             si, di, rb0, rb1, rcp_v, acc_s, gs0, gs1, ss0, ss1):
    c = lax.axis_index("c")
    s = lax.axis_index("s")
    row0 = s * ROWS_PER_TILE
    is_tail = s == NS - 1
    rb = (rb0, rb1)
    gs = (gs0, gs1)
    ss = (ss0, ss1)

    ones16 = jnp.full((L,), 1.0, jnp.float32)

    @pl.loop(0, CHUNK)
    def _(i):
        for k in range(D // L):
            rb0[i, pl.ds(k * L, L)] = ones16

    def zero_acc():
        pltpu.sync_copy(z128, acc_s.at[pl.ds(row0, ROWS_PER_TILE)])

        @pl.when(is_tail)
        def _():
            pltpu.sync_copy(z128.at[pl.ds(0, TAIL_ROWS)],
                            acc_s.at[pl.ds(NS * ROWS_PER_TILE, TAIL_ROWS)])

    zero_acc()
    plsc.subcore_barrier()

    def batch_a(dst2, i, nchunks):
        row = 8 * s + (BPC * NS) * i
        pltpu.sync_copy(dst2.at[pl.ds(row, nchunks)], di.at[pl.ds(0, nchunks)])
        copies = [pltpu.async_copy(rb0, acc_s.at[di.at[j]], ss0, add=True)
                  for j in range(nchunks)]
        for cp in copies:
            cp.wait()

    def phase_a(dst2):
        @pl.loop(0, NFULL)
        def _(i):
            batch_a(dst2, i, BPC)

        @pl.when(s < 8)
        def _():
            batch_a(dst2, NFULL, BPC)

        @pl.when(s == 8)
        def _():
            batch_a(dst2, NFULL, 4)

    @pl.when(c == 0)
    def _():
        phase_a(dst_d)

    @pl.when(c == 1)
    def _():
        phase_a(dst_t)

    plsc.subcore_barrier()

    def keep_rcp(nrows, src_row, dst_el):
        pltpu.sync_copy(acc_s.at[pl.ds(src_row, nrows)],
                        rb1.at[pl.ds(0, nrows)])

        @pl.loop(0, nrows)
        def _(i):
            deg = rb1[i, pl.ds(0, L)]
            rcp_v[pl.ds(dst_el + i * L, L)] = 1.0 / jnp.maximum(deg, 1.0)

    for local, nrows in BLOCKS:
        keep_rcp(nrows, row0 + local, local * L)

    @pl.when(is_tail)
    def _():
        keep_rcp(TAIL_ROWS, NS * ROWS_PER_TILE, ROWS_PER_TILE * L)

    plsc.subcore_barrier()
    zero_acc()
    plsc.subcore_barrier()

    def batch_b(x_hbm, src2, dst2, i, nchunks):
        row = 8 * s + (BPC * NS) * i
        pltpu.sync_copy(src2.at[pl.ds(row, nchunks)], si.at[pl.ds(0, nchunks)])
        pltpu.sync_copy(dst2.at[pl.ds(row, nchunks)], di.at[pl.ds(0, nchunks)])
        gcp = [None] * nchunks
        scp = [None] * nchunks
        gcp[0] = pltpu.async_copy(x_hbm.at[si.at[0]], rb[0], gs[0])
        for j in range(nchunks):
            gcp[j].wait()
            if j >= 1:
                scp[j - 1].wait()
            if j + 1 < nchunks:
                gcp[j + 1] = pltpu.async_copy(
                    x_hbm.at[si.at[j + 1]], rb[(j + 1) % 2], gs[(j + 1) % 2])
            scp[j] = pltpu.async_copy(
                rb[j % 2], acc_s.at[di.at[j]], ss[j % 2], add=True)
        scp[nchunks - 1].wait()

    def phase_b(x_hbm, src2, dst2):
        @pl.loop(0, NFULL)
        def _(i):
            batch_b(x_hbm, src2, dst2, i, BPC)

        @pl.when(s < 8)
        def _():
            batch_b(x_hbm, src2, dst2, NFULL, BPC)

        @pl.when(s == 8)
        def _():
            batch_b(x_hbm, src2, dst2, NFULL, 4)

    @pl.when(c == 0)
    def _():
        phase_b(x_d, src_d, dst_d)

    @pl.when(c == 1)
    def _():
        phase_b(x_t, src_t, dst_t)

    plsc.subcore_barrier()

    def flush_block(mean_out, nrows, src_row, rcp_el):
        pltpu.sync_copy(acc_s.at[pl.ds(src_row, nrows)],
                        rb0.at[pl.ds(0, nrows)])

        @pl.loop(0, nrows)
        def _(i):
            rcp = rcp_v[pl.ds(rcp_el + i * L, L)]
            for k in range(D // L):
                rb0[i, pl.ds(k * L, L)] = rb0[i, pl.ds(k * L, L)] * rcp

        pltpu.sync_copy(rb0.at[pl.ds(0, nrows)],
                        mean_out.at[pl.ds(src_row, nrows)])

    def flush(mean_out):
        for local, nrows in BLOCKS:
            flush_block(mean_out, nrows, row0 + local, local * L)

        @pl.when(is_tail)
        def _():
            flush_block(mean_out, TAIL_ROWS, NS * ROWS_PER_TILE,
                        ROWS_PER_TILE * L)

    @pl.when(c == 0)
    def _():
        flush(mean_d_out)

    @pl.when(c == 1)
    def _():
        flush(mean_t_out)


_sc_aggregate = pl.kernel(
    _sc_body,
    out_type=(
        jax.ShapeDtypeStruct((N, D), jnp.float32),
        jax.ShapeDtypeStruct((N, D), jnp.float32),
    ),
    mesh=plsc.VectorSubcoreMesh(core_axis_name="c", subcore_axis_name="s"),
    scratch_types=[
        pltpu.VMEM((BPC, CHUNK), jnp.int32),
        pltpu.VMEM((BPC, CHUNK), jnp.int32),
        pltpu.VMEM((CHUNK, D), jnp.float32),
        pltpu.VMEM((CHUNK, D), jnp.float32),
        pltpu.VMEM(((ROWS_PER_TILE + TAIL_ROWS) * L,), jnp.float32),
        pltpu.VMEM_SHARED((N, D), jnp.float32),
        pltpu.SemaphoreType.DMA,
        pltpu.SemaphoreType.DMA,
        pltpu.SemaphoreType.DMA,
        pltpu.SemaphoreType.DMA,
    ],
)


def _finish_body(x_ref, mean_ref, m_ref, w_ref, b_ref, o_ref):
    h = jnp.dot(mean_ref[...], w_ref[...],
                preferred_element_type=jnp.float32) + b_ref[...]
    h = jnp.maximum(h, 0.0)
    o_ref[...] = jnp.where(m_ref[...] != 0.0, x_ref[...], h)


def _finish(x, mean, m1, W, b2):
    R = 1000
    return pl.pallas_call(
        _finish_body,
        grid=(N // R,),
        in_specs=[
            pl.BlockSpec((R, D), lambda i: (i, 0)),
            pl.BlockSpec((R, D), lambda i: (i, 0)),
            pl.BlockSpec((R, 1), lambda i: (i, 0)),
            pl.BlockSpec((D, D), lambda i: (0, 0)),
            pl.BlockSpec((1, D), lambda i: (0, 0)),
        ],
        out_specs=pl.BlockSpec((R, D), lambda i: (i, 0)),
        out_shape=jax.ShapeDtypeStruct((N, D), jnp.float32),
    )(x, mean, m1, W, b2)


def kernel(x_drug, x_target, edge_index_dd, edge_index_tt,
           W_drug, b_drug, W_target, b_target):
    src_d = edge_index_dd[0].reshape(NROWS, CHUNK)
    dst_d = edge_index_dd[1].reshape(NROWS, CHUNK)
    src_t = edge_index_tt[0].reshape(NROWS, CHUNK)
    dst_t = edge_index_tt[1].reshape(NROWS, CHUNK)

    z128 = jnp.zeros((ROWS_PER_TILE, D), jnp.float32)

    mean_d, mean_t = _sc_aggregate(
        x_drug, x_target, src_d, dst_d, src_t, dst_t, z128)

    u_d = jax.random.uniform(jax.random.key(42), (N,), dtype=jnp.float32)
    u_t = jax.random.uniform(jax.random.key(43), (N,), dtype=jnp.float32)
    m_d = (u_d < 0.5).astype(jnp.float32).reshape(N, 1)
    m_t = (u_t < 0.5).astype(jnp.float32).reshape(N, 1)

    z_drug = _finish(x_drug, mean_d, m_d, W_drug, b_drug.reshape(1, D))
    z_target = _finish(x_target, mean_t, m_t, W_target, b_target.reshape(1, D))
    return (z_drug, z_target)
